# baseline (speedup 1.0000x reference)
import functools

import jax
import jax.numpy as jnp
from jax import lax
from jax.experimental import pallas as pl
from jax.experimental.pallas import tpu as pltpu
from jax.experimental.pallas import tpu_sc as plsc

VOCAB = 1000000
DIM = 16
LABELS = 32
N = 3276800

NC = 2
NS = 16
NW = NC * NS

C = 2048
K = C // 128

BT = 16384
MB = BT // 16
NB = N // BT

NBPS = (8, 24, 56, 56, 56)
assert sum(NBPS) == NB


def _sc_gather(idx4d, table, nchunk, np_tok):
    mesh = plsc.VectorSubcoreMesh(core_axis_name="c", subcore_axis_name="s")
    bpw = np_tok // NW

    @functools.partial(
        pl.kernel,
        mesh=mesh,
        out_type=jax.ShapeDtypeStruct((np_tok // 16, 128), jnp.int32),
        scratch_types=[
            pltpu.VMEM((K, 128), jnp.int32),
            pltpu.VMEM((C, DIM), jnp.float32),
            pltpu.VMEM((C // 16, 128), jnp.int32),
            pltpu.SemaphoreType.DMA,
        ],
        compiler_params=pltpu.CompilerParams(use_tc_tiling_on_sc=False),
    )
    def k(idx_hbm, table_hbm, out_hbm, idx_v, rows_v, pk_v, sem):
        wid = lax.axis_index("s") * NC + lax.axis_index("c")
        half = jnp.int32(0x7FFF)
        one = jnp.int32(1)
        himask = jnp.int32(-65536)

        def body(i, carry):
            pltpu.sync_copy(idx_hbm.at[wid, i], idx_v)
            cps = [
                pltpu.async_copy(
                    table_hbm.at[idx_v.at[j]],
                    rows_v.at[pl.ds(j * 128, 128)],
                    sem,
                )
                for j in range(K)
            ]
            for cp in cps:
                cp.wait()

            def pbody(r, carry2):
                for u in range(8):
                    x = rows_v[r * 16 + 2 * u]
                    y = rows_v[r * 16 + 2 * u + 1]
                    bx = lax.bitcast_convert_type(x, jnp.int32)
                    by = lax.bitcast_convert_type(y, jnp.int32)
                    rx = bx + half + (lax.shift_right_logical(bx, 16) & one)
                    ry = by + half + (lax.shift_right_logical(by, 16) & one)
                    z = lax.shift_right_logical(rx, 16) | (ry & himask)
                    pk_v[r, pl.ds(u * 16, 16)] = z
                return carry2

            lax.fori_loop(0, C // 16, pbody, 0)
            pltpu.sync_copy(
                pk_v, out_hbm.at[pl.ds((wid * bpw + i * C) // 16, C // 16)])
            return carry

        lax.fori_loop(0, nchunk, body, 0)

    return k(idx4d, table)


def _tc_dense(pk, W, b, eye, nbp, blk0, carry):
    def body(pk_ref, w_ref, b_ref, eye_ref, *rest):
        scT_ref, embT_ref = rest[-2], rest[-1]
        e = pk_ref[...]
        ex = lax.bitcast_convert_type(
            lax.shift_left(e, 16), jnp.float32)
        ey = lax.bitcast_convert_type(
            e & jnp.int32(-65536), jnp.float32)
        ident = eye_ref[...]
        slabs = []
        for g in range(8):
            px = ex[:, 16 * g:16 * (g + 1)]
            py = ey[:, 16 * g:16 * (g + 1)]
            slabs.append(lax.dot_general(
                ident, px, (((1,), (1,)), ((), ())),
                preferred_element_type=jnp.float32))
            slabs.append(lax.dot_general(
                ident, py, (((1,), (1,)), ((), ())),
                preferred_element_type=jnp.float32))
        eT = jnp.concatenate(slabs, axis=1)
        embT_ref[...] = eT
        t = lax.dot_general(
            w_ref[...], eT, (((0,), (0,)), ((), ())),
            preferred_element_type=jnp.float32) + b_ref[...]
        m = jnp.max(t, axis=0, keepdims=True)
        s = t - m
        scT_ref[...] = s - jnp.log(jnp.sum(jnp.exp(s), axis=0, keepdims=True))

    in_specs = [
        pl.BlockSpec((MB, 128), lambda i: (i, 0)),
        pl.BlockSpec((DIM, LABELS), lambda i: (0, 0)),
        pl.BlockSpec((LABELS, 1), lambda i: (0, 0)),
        pl.BlockSpec((DIM, DIM), lambda i: (0, 0)),
    ]
    operands = [pk, W, b, eye]
    kwargs = {}
    if carry is not None:
        in_specs += [
            pl.BlockSpec(memory_space=pl.ANY),
            pl.BlockSpec(memory_space=pl.ANY),
        ]
        operands += [carry[0], carry[1]]
        kwargs["input_output_aliases"] = {4: 0, 5: 1}

    return pl.pallas_call(
        body,
        grid=(nbp,),
        in_specs=in_specs,
        out_specs=[
            pl.BlockSpec((LABELS, BT), lambda i, o=blk0: (0, o + i)),
            pl.BlockSpec((DIM, BT), lambda i, o=blk0: (0, o + i)),
        ],
        out_shape=[
            jax.ShapeDtypeStruct((LABELS, N), jnp.float32),
            jax.ShapeDtypeStruct((DIM, N), jnp.float32),
        ],
        **kwargs,
    )(*operands)


def kernel(sentence, table, W, b):
    eye = jnp.eye(DIM, dtype=jnp.float32)
    b2 = b.reshape(LABELS, 1)
    pks = []
    tok0 = 0
    for nbp in NBPS:
        np_tok = nbp * BT
        sl = lax.slice_in_dim(sentence, tok0, tok0 + np_tok)
        idx_fed = sl.reshape(nbp, 16, MB).swapaxes(1, 2)
        nchunk = np_tok // NW // C
        idx4d = idx_fed.reshape(NW, nchunk, K, 128)
        pks.append(_sc_gather(idx4d, table, nchunk, np_tok))
        tok0 += np_tok
    carry = None
    blk0 = 0
    for nbp, pk in zip(NBPS, pks):
        carry = _tc_dense(pk, W, b2, eye, nbp, blk0, carry)
        blk0 += nbp
    scoresT, embT = carry
    return scoresT.T, embT.T


# speedup vs baseline: 1.0184x; 1.0184x over previous
import functools

import jax
import jax.numpy as jnp
from jax import lax
from jax.experimental import pallas as pl
from jax.experimental.pallas import tpu as pltpu
from jax.experimental.pallas import tpu_sc as plsc

VOCAB = 1000000
DIM = 16
LABELS = 32
N = 3276800

NC = 2
NS = 16
NW = NC * NS

C = 2048
K = C // 128

BT = 16384
MB = BT // 16
NB = N // BT

NBPS = (40, 40, 40, 40, 40)
assert sum(NBPS) == NB


def _sc_gather(idx4d, table, nchunk, np_tok):
    mesh = plsc.VectorSubcoreMesh(core_axis_name="c", subcore_axis_name="s")
    bpw = np_tok // NW

    @functools.partial(
        pl.kernel,
        mesh=mesh,
        out_type=jax.ShapeDtypeStruct((np_tok // 16, 128), jnp.int32),
        scratch_types=[
            pltpu.VMEM((K, 128), jnp.int32),
            pltpu.VMEM((C, DIM), jnp.float32),
            pltpu.VMEM((C // 16, 128), jnp.int32),
            pltpu.SemaphoreType.DMA,
        ],
        compiler_params=pltpu.CompilerParams(use_tc_tiling_on_sc=False),
    )
    def k(idx_hbm, table_hbm, out_hbm, idx_v, rows_v, pk_v, sem):
        wid = lax.axis_index("s") * NC + lax.axis_index("c")
        half = jnp.int32(0x7FFF)
        one = jnp.int32(1)
        himask = jnp.int32(-65536)

        def body(i, carry):
            pltpu.sync_copy(idx_hbm.at[wid, i], idx_v)
            cps = [
                pltpu.async_copy(
                    table_hbm.at[idx_v.at[j]],
                    rows_v.at[pl.ds(j * 128, 128)],
                    sem,
                )
                for j in range(K)
            ]
            for cp in cps:
                cp.wait()

            def pbody(r, carry2):
                for u in range(8):
                    x = rows_v[r * 16 + 2 * u]
                    y = rows_v[r * 16 + 2 * u + 1]
                    bx = lax.bitcast_convert_type(x, jnp.int32)
                    by = lax.bitcast_convert_type(y, jnp.int32)
                    rx = bx + half + (lax.shift_right_logical(bx, 16) & one)
                    ry = by + half + (lax.shift_right_logical(by, 16) & one)
                    z = lax.shift_right_logical(rx, 16) | (ry & himask)
                    pk_v[r, pl.ds(u * 16, 16)] = z
                return carry2

            lax.fori_loop(0, C // 16, pbody, 0)
            pltpu.sync_copy(
                pk_v, out_hbm.at[pl.ds((wid * bpw + i * C) // 16, C // 16)])
            return carry

        lax.fori_loop(0, nchunk, body, 0)

    return k(idx4d, table)


def _tc_dense(pk, W, b, eye, nbp, blk0, carry):
    def body(pk_ref, w_ref, b_ref, eye_ref, *rest):
        scT_ref, embT_ref = rest[-2], rest[-1]
        e = pk_ref[...]
        ex = lax.bitcast_convert_type(
            lax.shift_left(e, 16), jnp.float32)
        ey = lax.bitcast_convert_type(
            e & jnp.int32(-65536), jnp.float32)
        ident = eye_ref[...]
        slabs = []
        for g in range(8):
            px = ex[:, 16 * g:16 * (g + 1)]
            py = ey[:, 16 * g:16 * (g + 1)]
            slabs.append(lax.dot_general(
                ident, px, (((1,), (1,)), ((), ())),
                preferred_element_type=jnp.float32))
            slabs.append(lax.dot_general(
                ident, py, (((1,), (1,)), ((), ())),
                preferred_element_type=jnp.float32))
        eT = jnp.concatenate(slabs, axis=1)
        embT_ref[...] = eT
        t = lax.dot_general(
            w_ref[...], eT, (((0,), (0,)), ((), ())),
            preferred_element_type=jnp.float32) + b_ref[...]
        m = jnp.max(t, axis=0, keepdims=True)
        s = t - m
        scT_ref[...] = s - jnp.log(jnp.sum(jnp.exp(s), axis=0, keepdims=True))

    in_specs = [
        pl.BlockSpec((MB, 128), lambda i: (i, 0)),
        pl.BlockSpec((DIM, LABELS), lambda i: (0, 0)),
        pl.BlockSpec((LABELS, 1), lambda i: (0, 0)),
        pl.BlockSpec((DIM, DIM), lambda i: (0, 0)),
    ]
    operands = [pk, W, b, eye]
    kwargs = {}
    if carry is not None:
        in_specs += [
            pl.BlockSpec(memory_space=pl.ANY),
            pl.BlockSpec(memory_space=pl.ANY),
        ]
        operands += [carry[0], carry[1]]
        kwargs["input_output_aliases"] = {4: 0, 5: 1}

    return pl.pallas_call(
        body,
        grid=(nbp,),
        in_specs=in_specs,
        out_specs=[
            pl.BlockSpec((LABELS, BT), lambda i, o=blk0: (0, o + i)),
            pl.BlockSpec((DIM, BT), lambda i, o=blk0: (0, o + i)),
        ],
        out_shape=[
            jax.ShapeDtypeStruct((LABELS, N), jnp.float32),
            jax.ShapeDtypeStruct((DIM, N), jnp.float32),
        ],
        **kwargs,
    )(*operands)


def kernel(sentence, table, W, b):
    eye = jnp.eye(DIM, dtype=jnp.float32)
    b2 = b.reshape(LABELS, 1)
    pks = []
    tok0 = 0
    for nbp in NBPS:
        np_tok = nbp * BT
        sl = lax.slice_in_dim(sentence, tok0, tok0 + np_tok)
        idx_fed = sl.reshape(nbp, 16, MB).swapaxes(1, 2)
        nchunk = np_tok // NW // C
        idx4d = idx_fed.reshape(NW, nchunk, K, 128)
        pks.append(_sc_gather(idx4d, table, nchunk, np_tok))
        tok0 += np_tok
    carry = None
    blk0 = 0
    for nbp, pk in zip(NBPS, pks):
        carry = _tc_dense(pk, W, b2, eye, nbp, blk0, carry)
        blk0 += nbp
    scoresT, embT = carry
    return scoresT.T, embT.T


# speedup vs baseline: 1.1706x; 1.1495x over previous
import functools

import jax
import jax.numpy as jnp
from jax import lax
from jax.experimental import pallas as pl
from jax.experimental.pallas import tpu as pltpu
from jax.experimental.pallas import tpu_sc as plsc

VOCAB = 1000000
DIM = 16
LABELS = 32
N = 3276800

NC = 2
NS = 16
NW = NC * NS

C = 2048
K = C // 128

BT = 16384
MB = BT // 16
NB = N // BT

NBPS = (40, 40, 40, 40, 40)
assert sum(NBPS) == NB


def _sc_gather(idx4d, table, nchunk, np_tok):
    mesh = plsc.VectorSubcoreMesh(core_axis_name="c", subcore_axis_name="s")
    bpw = np_tok // NW

    @functools.partial(
        pl.kernel,
        mesh=mesh,
        out_type=jax.ShapeDtypeStruct((np_tok // 16, 128), jnp.int32),
        scratch_types=[
            pltpu.VMEM((K, 128), jnp.int32),
            pltpu.VMEM((K, 128), jnp.int32),
            pltpu.VMEM((C, DIM), jnp.float32),
            pltpu.VMEM((C, DIM), jnp.float32),
            pltpu.VMEM((C // 16, 128), jnp.int32),
            pltpu.SemaphoreType.DMA,
            pltpu.SemaphoreType.DMA,
        ],
        compiler_params=pltpu.CompilerParams(use_tc_tiling_on_sc=False),
    )
    def k(idx_hbm, table_hbm, out_hbm, idx_v0, idx_v1, rows_v0, rows_v1,
          pk_v, sem0, sem1):
        wid = lax.axis_index("s") * NC + lax.axis_index("c")
        half = jnp.int32(0x7FFF)
        one = jnp.int32(1)
        himask = jnp.int32(-65536)
        bufs = ((idx_v0, rows_v0, sem0), (idx_v1, rows_v1, sem1))

        def fire(i, par):
            idx_v, rows_v, sem = bufs[par]
            pltpu.sync_copy(idx_hbm.at[wid, i], idx_v)
            return [
                pltpu.async_copy(
                    table_hbm.at[idx_v.at[j]],
                    rows_v.at[pl.ds(j * 128, 128)],
                    sem,
                )
                for j in range(K)
            ]

        def drain(par):
            idx_v, rows_v, sem = bufs[par]
            for j in range(K):
                pltpu.make_async_copy(
                    table_hbm.at[idx_v.at[j]],
                    rows_v.at[pl.ds(j * 128, 128)],
                    sem,
                ).wait()

        def pack_out(i, par):
            rows_v = bufs[par][1]

            def pbody(r, carry2):
                for u in range(8):
                    x = rows_v[r * 16 + 2 * u]
                    y = rows_v[r * 16 + 2 * u + 1]
                    bx = lax.bitcast_convert_type(x, jnp.int32)
                    by = lax.bitcast_convert_type(y, jnp.int32)
                    rx = bx + half + (lax.shift_right_logical(bx, 16) & one)
                    ry = by + half + (lax.shift_right_logical(by, 16) & one)
                    z = lax.shift_right_logical(rx, 16) | (ry & himask)
                    pk_v[r, pl.ds(u * 16, 16)] = z
                return carry2

            lax.fori_loop(0, C // 16, pbody, 0)
            pltpu.sync_copy(
                pk_v, out_hbm.at[pl.ds((wid * bpw + i * C) // 16, C // 16)])

        fire(0, 0)

        def body(i2, carry):
            i = i2 * 2
            drain(0)
            fire(i + 1, 1)
            pack_out(i, 0)
            drain(1)

            @pl.when(i + 2 < nchunk)
            def _():
                fire(i + 2, 0)

            pack_out(i + 1, 1)
            return carry

        lax.fori_loop(0, nchunk // 2, body, 0)

    return k(idx4d, table)


def _tc_dense(pk, W, b, eye, nbp, blk0, carry):
    def body(pk_ref, w_ref, b_ref, eye_ref, *rest):
        scT_ref, embT_ref = rest[-2], rest[-1]
        e = pk_ref[...]
        ex = lax.bitcast_convert_type(
            lax.shift_left(e, 16), jnp.float32)
        ey = lax.bitcast_convert_type(
            e & jnp.int32(-65536), jnp.float32)
        ident = eye_ref[...]
        slabs = []
        for g in range(8):
            px = ex[:, 16 * g:16 * (g + 1)]
            py = ey[:, 16 * g:16 * (g + 1)]
            slabs.append(lax.dot_general(
                ident, px, (((1,), (1,)), ((), ())),
                preferred_element_type=jnp.float32))
            slabs.append(lax.dot_general(
                ident, py, (((1,), (1,)), ((), ())),
                preferred_element_type=jnp.float32))
        eT = jnp.concatenate(slabs, axis=1)
        embT_ref[...] = eT
        t = lax.dot_general(
            w_ref[...], eT, (((0,), (0,)), ((), ())),
            preferred_element_type=jnp.float32) + b_ref[...]
        m = jnp.max(t, axis=0, keepdims=True)
        s = t - m
        scT_ref[...] = s - jnp.log(jnp.sum(jnp.exp(s), axis=0, keepdims=True))

    in_specs = [
        pl.BlockSpec((MB, 128), lambda i: (i, 0)),
        pl.BlockSpec((DIM, LABELS), lambda i: (0, 0)),
        pl.BlockSpec((LABELS, 1), lambda i: (0, 0)),
        pl.BlockSpec((DIM, DIM), lambda i: (0, 0)),
    ]
    operands = [pk, W, b, eye]
    kwargs = {}
    if carry is not None:
        in_specs += [
            pl.BlockSpec(memory_space=pl.ANY),
            pl.BlockSpec(memory_space=pl.ANY),
        ]
        operands += [carry[0], carry[1]]
        kwargs["input_output_aliases"] = {4: 0, 5: 1}

    return pl.pallas_call(
        body,
        grid=(nbp,),
        in_specs=in_specs,
        out_specs=[
            pl.BlockSpec((LABELS, BT), lambda i, o=blk0: (0, o + i)),
            pl.BlockSpec((DIM, BT), lambda i, o=blk0: (0, o + i)),
        ],
        out_shape=[
            jax.ShapeDtypeStruct((LABELS, N), jnp.float32),
            jax.ShapeDtypeStruct((DIM, N), jnp.float32),
        ],
        **kwargs,
    )(*operands)


def kernel(sentence, table, W, b):
    eye = jnp.eye(DIM, dtype=jnp.float32)
    b2 = b.reshape(LABELS, 1)
    pks = []
    tok0 = 0
    for nbp in NBPS:
        np_tok = nbp * BT
        sl = lax.slice_in_dim(sentence, tok0, tok0 + np_tok)
        idx_fed = sl.reshape(nbp, 16, MB).swapaxes(1, 2)
        nchunk = np_tok // NW // C
        idx4d = idx_fed.reshape(NW, nchunk, K, 128)
        pks.append(_sc_gather(idx4d, table, nchunk, np_tok))
        tok0 += np_tok
    carry = None
    blk0 = 0
    for nbp, pk in zip(NBPS, pks):
        carry = _tc_dense(pk, W, b2, eye, nbp, blk0, carry)
        blk0 += nbp
    scoresT, embT = carry
    return scoresT.T, embT.T
